# baseline (speedup 1.0000x reference)
import functools

import jax
import jax.numpy as jnp
from jax import lax
from jax.experimental import pallas as pl
from jax.experimental.pallas import tpu as pltpu
from jax.experimental.pallas import tpu_sc as plsc

N = 10000
NPAD = 10240
E = 160000
EPAD = 163840
EP = 20000
EPPAD = 20480
NTILES = 32
NSUB = 16
CHUNK = 128
NCH = EPAD // (NTILES * CHUNK)
ROWS_PER_SUB = NPAD // NSUB
NFLUSH = ROWS_PER_SUB // CHUNK
BM = 1000
F32 = jnp.float32


def _mesh():
  return plsc.VectorSubcoreMesh(core_axis_name="c", subcore_axis_name="s")


def _deg_call(src_idx, dst_idx):
  @functools.partial(
      pl.kernel,
      mesh=_mesh(),
      out_type=jax.ShapeDtypeStruct((2, 2, NPAD, 128), F32),
      scratch_types=[
          pltpu.VMEM((NCH, CHUNK), jnp.int32),
          pltpu.VMEM((CHUNK, 128), F32),
          pltpu.VMEM_SHARED((NPAD, 128), F32),
          pltpu.SemaphoreType.DMA,
      ],
  )
  def deg(src_hbm, dst_hbm, out, didx, ones_v, acc, sem):
    c = lax.axis_index("c")
    s = lax.axis_index("s")
    wid = c * NSUB + s

    for d, idx_hbm in ((0, src_hbm), (1, dst_hbm)):
      def zfill(i, _):
        for j in range(128 // 16):
          ones_v[i, pl.ds(16 * j, 16)] = jnp.zeros((16,), F32)
        return 0
      lax.fori_loop(0, CHUNK, zfill, 0)
      for f in range(NFLUSH):
        r0 = s * ROWS_PER_SUB + f * CHUNK
        pltpu.sync_copy(ones_v, acc.at[pl.ds(r0, CHUNK)])

      def ofill(i, _):
        for j in range(128 // 16):
          ones_v[i, pl.ds(16 * j, 16)] = jnp.ones((16,), F32)
        return 0
      lax.fori_loop(0, CHUNK, ofill, 0)
      pltpu.sync_copy(idx_hbm.at[wid], didx)
      plsc.subcore_barrier()

      def body(k, _):
        pltpu.async_copy(ones_v, acc.at[didx.at[k]], sem, add=True)
        return 0
      lax.fori_loop(0, NCH, body, 0)

      def drain(k, _):
        pltpu.make_async_copy(ones_v, acc.at[didx.at[0]], sem).wait()
        return 0
      lax.fori_loop(0, NCH, drain, 0)
      plsc.subcore_barrier()

      for f in range(NFLUSH):
        r0 = s * ROWS_PER_SUB + f * CHUNK
        pltpu.sync_copy(acc.at[pl.ds(r0, CHUNK)], ones_v)
        pltpu.sync_copy(ones_v, out.at[d, c, pl.ds(r0, CHUNK)])

  return deg(src_idx, dst_idx)


def _agg_call(hw, src_idx, dst_idx, NB, FB):
  fo = NB * FB

  @functools.partial(
      pl.kernel,
      mesh=_mesh(),
      out_type=jax.ShapeDtypeStruct((2, NPAD, fo), F32),
      scratch_types=[
          pltpu.VMEM((NCH, CHUNK), jnp.int32),
          pltpu.VMEM((NCH, CHUNK), jnp.int32),
          pltpu.VMEM((CHUNK, FB), F32),
          pltpu.VMEM((CHUNK, FB), F32),
          pltpu.VMEM_SHARED((NPAD, FB), F32),
          pltpu.SemaphoreType.DMA,
          pltpu.SemaphoreType.DMA,
          pltpu.SemaphoreType.DMA,
          pltpu.SemaphoreType.DMA,
      ],
  )
  def agg(hw_hbm, src_hbm, dst_hbm, out, sidx, didx, buf0, buf1, acc,
          sem0, sem1, ssem0, ssem1):
    c = lax.axis_index("c")
    s = lax.axis_index("s")
    wid = c * NSUB + s
    bufs = (buf0, buf1)
    sems = (sem0, sem1)
    ssems = (ssem0, ssem1)

    def fill(i, _):
      for j in range(FB // 16):
        buf0[i, pl.ds(16 * j, 16)] = jnp.zeros((16,), F32)
      return 0

    pltpu.sync_copy(src_hbm.at[wid], sidx)
    pltpu.sync_copy(dst_hbm.at[wid], didx)

    for b in range(NB):
      lax.fori_loop(0, CHUNK, fill, 0)
      for f in range(NFLUSH):
        pltpu.sync_copy(buf0, acc.at[pl.ds(s * ROWS_PER_SUB + f * CHUNK, CHUNK)])
      plsc.subcore_barrier()

      tbl = hw_hbm.at[b]
      pltpu.make_async_copy(tbl.at[sidx.at[0]], buf0, sem0).start()

      def body(t, _):
        for par in range(2):
          k = 2 * t + par
          nk = k + 1
          other = (par + 1) % 2

          pltpu.make_async_copy(tbl.at[sidx.at[k]], bufs[par], sems[par]).wait()
          pltpu.async_copy(bufs[par], acc.at[didx.at[k]], ssems[par], add=True)

          @pl.when(k >= 1)
          def _():
            pltpu.make_async_copy(
                bufs[other], acc.at[didx.at[k]], ssems[other]
            ).wait()

          @pl.when(nk < NCH)
          def _():
            pltpu.make_async_copy(
                tbl.at[sidx.at[nk]], bufs[other], sems[other]
            ).start()
        return 0
      lax.fori_loop(0, NCH // 2, body, 0)
      pltpu.make_async_copy(
          bufs[(NCH - 1) % 2], acc.at[didx.at[NCH - 1]], ssems[(NCH - 1) % 2]
      ).wait()
      plsc.subcore_barrier()

      for f in range(NFLUSH):
        r0 = s * ROWS_PER_SUB + f * CHUNK
        pltpu.sync_copy(acc.at[pl.ds(r0, CHUNK)], buf0)
        pltpu.sync_copy(buf0, out.at[c, pl.ds(r0, CHUNK), pl.ds(b * FB, FB)])
      if b + 1 < NB:
        plsc.subcore_barrier()

  return agg(hw, src_idx, dst_idx)


def _score_gather_call(ab, idx4):
  NCH_S = EPPAD // (NTILES * CHUNK)

  @functools.partial(
      pl.kernel,
      mesh=_mesh(),
      out_type=jax.ShapeDtypeStruct((4, EPPAD, 128), F32),
      scratch_types=[
          pltpu.VMEM((NCH_S, CHUNK), jnp.int32),
          pltpu.VMEM((CHUNK, 128), F32),
          pltpu.VMEM((CHUNK, 128), F32),
          pltpu.SemaphoreType.DMA,
          pltpu.SemaphoreType.DMA,
      ],
  )
  def sg(ab_hbm, idx_hbm, out, idxv, buf0, buf1, sem0, sem1):
    c = lax.axis_index("c")
    s = lax.axis_index("s")
    wid = c * NSUB + s
    bufs = (buf0, buf1)
    sems = (sem0, sem1)
    for t in range(4):
      tbl = ab_hbm.at[t % 2]
      pltpu.sync_copy(idx_hbm.at[t, wid], idxv)
      pltpu.make_async_copy(tbl.at[idxv.at[0]], buf0, sem0).start()
      for k in range(NCH_S):
        if k + 1 < NCH_S:
          pltpu.make_async_copy(
              tbl.at[idxv.at[k + 1]], bufs[(k + 1) % 2], sems[(k + 1) % 2]
          ).start()
        pltpu.make_async_copy(tbl.at[idxv.at[k]], bufs[k % 2], sems[k % 2]).wait()
        pltpu.sync_copy(
            bufs[k % 2], out.at[t, pl.ds(wid * 640 + k * CHUNK, CHUNK)]
        )

  return sg(ab, idx4)


def _isd_call(deg):
  def body(d_ref, o_ref):
    d = d_ref[...]
    o_ref[...] = lax.rsqrt(jnp.maximum(d[:, 0] + d[:, 1], 1.0))

  return pl.pallas_call(
      body,
      grid=(1,),
      in_specs=[pl.BlockSpec((2, 2, 80, 128), lambda i: (0, 0, 0, 0))],
      out_specs=pl.BlockSpec((2, 80, 128), lambda i: (0, 0, 0)),
      out_shape=jax.ShapeDtypeStruct((2, 80, 128), F32),
  )(deg)


def _mm_first_call(x, isd_out, W, NB, FB):
  fi = x.shape[1]

  def body(x_ref, so_ref, w_ref, o_ref):
    o_ref[0] = jnp.dot(x_ref[...], w_ref[...],
                       preferred_element_type=F32) * so_ref[...]

  return pl.pallas_call(
      body,
      grid=(N // BM, NB),
      in_specs=[
          pl.BlockSpec((BM, fi), lambda i, j: (i, 0)),
          pl.BlockSpec((BM, 1), lambda i, j: (i, 0)),
          pl.BlockSpec((fi, FB), lambda i, j: (0, j)),
      ],
      out_specs=pl.BlockSpec((1, BM, FB), lambda i, j: (j, i, 0)),
      out_shape=jax.ShapeDtypeStruct((NB, N, FB), F32),
  )(x, isd_out, W)


def _mm_mid_call(acc, isd_in, b, isd_out, W, NB, FB, relu):
  fi = acc.shape[2]

  def body(a_ref, si_ref, b_ref, so_ref, w_ref, o_ref, a_s):
    j = pl.program_id(1)

    @pl.when(j == 0)
    def _():
      h = (a_ref[0] + a_ref[1]) * si_ref[...] + b_ref[...]
      if relu:
        h = jnp.maximum(h, 0.0)
      a_s[...] = h

    o_ref[0] = jnp.dot(a_s[...], w_ref[...],
                       preferred_element_type=F32) * so_ref[...]

  return pl.pallas_call(
      body,
      grid=(N // BM, NB),
      in_specs=[
          pl.BlockSpec((2, BM, fi), lambda i, j: (0, i, 0)),
          pl.BlockSpec((BM, 1), lambda i, j: (i, 0)),
          pl.BlockSpec((1, fi), lambda i, j: (0, 0)),
          pl.BlockSpec((BM, 1), lambda i, j: (i, 0)),
          pl.BlockSpec((fi, FB), lambda i, j: (0, j)),
      ],
      out_specs=pl.BlockSpec((1, BM, FB), lambda i, j: (j, i, 0)),
      out_shape=jax.ShapeDtypeStruct((NB, N, FB), F32),
      scratch_shapes=[pltpu.VMEM((BM, fi), F32)],
  )(acc, isd_in, b, isd_out, W)


def _mm_ab_call(acc, isd_in, b4, Wp, bb):
  def body(a_ref, si_ref, b_ref, w_ref, bb_ref, o_ref, a_s):
    j = pl.program_id(1)

    @pl.when(j == 0)
    def _():
      a_s[...] = (a_ref[0, :, :64] + a_ref[1, :, :64]) * si_ref[...] + b_ref[...]

    o_ref[0] = jnp.dot(a_s[...], w_ref[0], preferred_element_type=F32) + bb_ref[0]

  return pl.pallas_call(
      body,
      grid=(N // BM, 2),
      in_specs=[
          pl.BlockSpec((2, BM, 128), lambda i, j: (0, i, 0)),
          pl.BlockSpec((BM, 1), lambda i, j: (i, 0)),
          pl.BlockSpec((1, 64), lambda i, j: (0, 0)),
          pl.BlockSpec((1, 64, 128), lambda i, j: (j, 0, 0)),
          pl.BlockSpec((1, 1, 128), lambda i, j: (j, 0, 0)),
      ],
      out_specs=pl.BlockSpec((1, BM, 128), lambda i, j: (j, i, 0)),
      out_shape=jax.ShapeDtypeStruct((2, N, 128), F32),
      scratch_shapes=[pltpu.VMEM((BM, 64), F32)],
  )(acc, isd_in, b4, Wp, bb)


def _score_mm_call(Z, Wp2, bp2):
  BM2 = 1024

  def body(z1_ref, z2_ref, w_ref, b_ref, o_ref):
    z = jnp.maximum(z1_ref[0, :, :64] + z2_ref[0, :, :64], 0.0)
    o_ref[0] = jnp.dot(z, w_ref[...], preferred_element_type=F32) + b_ref[...]

  return pl.pallas_call(
      body,
      grid=(2, EPPAD // BM2),
      in_specs=[
          pl.BlockSpec((1, BM2, 128), lambda k, i: (2 * k, i, 0)),
          pl.BlockSpec((1, BM2, 128), lambda k, i: (2 * k + 1, i, 0)),
          pl.BlockSpec((64, 1), lambda k, i: (0, 0)),
          pl.BlockSpec((1, 1), lambda k, i: (0, 0)),
      ],
      out_specs=pl.BlockSpec((1, BM2, 1), lambda k, i: (k, i, 0)),
      out_shape=jax.ShapeDtypeStruct((2, EPPAD, 1), F32),
  )(Z, Z, Wp2, bp2)


def _pad_gather(idx, total):
  pad = total - idx.shape[0]
  fill = (jnp.arange(pad, dtype=jnp.int32) * 97) % N
  return jnp.concatenate([idx.astype(jnp.int32), fill])


def _pad_trash(idx, total):
  pad = total - idx.shape[0]
  fill = N + (jnp.arange(pad, dtype=jnp.int32) % (NPAD - N))
  return jnp.concatenate([idx.astype(jnp.int32), fill])


def kernel(x, edge_index, pos_edge_index, neg_edge_index,
           W1, b1, W2, b2, W3, b3, W4, b4, Wp1, bp1, Wp2, bp2):
  src, dst = edge_index[0], edge_index[1]
  src_g = _pad_gather(src, EPAD).reshape(NTILES, NCH, CHUNK)
  dst_s = _pad_trash(dst, EPAD).reshape(NTILES, NCH, CHUNK)
  src_t = _pad_trash(src, EPAD).reshape(NTILES, NCH, CHUNK)

  deg = _deg_call(src_t, dst_s)
  isd = _isd_call(deg[..., 0].reshape(2, 2, 80, 128))
  isd_out = isd[0].reshape(NPAD, 1)[:N]
  isd_in = isd[1].reshape(NPAD, 1)[:N]

  hw1 = _mm_first_call(x, isd_out, W1, 4, 128)
  acc1 = _agg_call(hw1, src_g, dst_s, 4, 128)
  hw2 = _mm_mid_call(acc1[:, :N], isd_in, b1.reshape(1, -1), isd_out, W2,
                     2, 128, True)
  acc2 = _agg_call(hw2, src_g, dst_s, 2, 128)
  hw3 = _mm_mid_call(acc2[:, :N], isd_in, b2.reshape(1, -1), isd_out, W3,
                     1, 128, True)
  acc3 = _agg_call(hw3, src_g, dst_s, 1, 128)
  W4p = jnp.pad(W4, ((0, 0), (0, 64)))
  hw4 = _mm_mid_call(acc3[:, :N], isd_in, b3.reshape(1, -1), isd_out, W4p,
                     1, 128, True)
  acc4 = _agg_call(hw4, src_g, dst_s, 1, 128)

  Wp = jnp.pad(Wp1.reshape(2, 64, 64), ((0, 0), (0, 0), (0, 64)))
  bb = jnp.pad(jnp.stack([bp1, jnp.zeros_like(bp1)]).reshape(2, 1, 64),
               ((0, 0), (0, 0), (0, 64)))
  AB = _mm_ab_call(acc4[:, :N], isd_in, b4.reshape(1, -1), Wp, bb)

  idx4 = jnp.stack([
      _pad_gather(pos_edge_index[0], EPPAD),
      _pad_gather(pos_edge_index[1], EPPAD),
      _pad_gather(neg_edge_index[0], EPPAD),
      _pad_gather(neg_edge_index[1], EPPAD),
  ]).reshape(4, NTILES, EPPAD // (NTILES * CHUNK), CHUNK)

  Z = _score_gather_call(AB, idx4)
  S = _score_mm_call(Z, Wp2, bp2.reshape(1, 1))
  return S[0, :EP], S[1, :EP]


# speedup vs baseline: 1.1022x; 1.1022x over previous
import functools

import jax
import jax.numpy as jnp
from jax import lax
from jax.experimental import pallas as pl
from jax.experimental.pallas import tpu as pltpu
from jax.experimental.pallas import tpu_sc as plsc

N = 10000
NPAD = 10240
E = 160000
EPAD = 163840
EP = 20000
EPPAD = 20480
NTILES = 32
NSUB = 16
CHUNK = 128
NCH = EPAD // (NTILES * CHUNK)
ROWS_PER_SUB = NPAD // NSUB
NFLUSH = ROWS_PER_SUB // CHUNK
BM = 1000
F32 = jnp.float32


def _mesh():
  return plsc.VectorSubcoreMesh(core_axis_name="c", subcore_axis_name="s")


def _deg_call(src_idx, dst_idx):
  @functools.partial(
      pl.kernel,
      mesh=_mesh(),
      out_type=jax.ShapeDtypeStruct((2, 2, NPAD, 128), F32),
      scratch_types=[
          pltpu.VMEM((NCH, CHUNK), jnp.int32),
          pltpu.VMEM((CHUNK, 128), F32),
          pltpu.VMEM_SHARED((NPAD, 128), F32),
          pltpu.SemaphoreType.DMA,
      ],
  )
  def deg(src_hbm, dst_hbm, out, didx, ones_v, acc, sem):
    c = lax.axis_index("c")
    s = lax.axis_index("s")
    wid = c * NSUB + s

    for d, idx_hbm in ((0, src_hbm), (1, dst_hbm)):
      def zfill(i, _):
        for j in range(128 // 16):
          ones_v[i, pl.ds(16 * j, 16)] = jnp.zeros((16,), F32)
        return 0
      lax.fori_loop(0, CHUNK, zfill, 0)
      for f in range(NFLUSH):
        r0 = s * ROWS_PER_SUB + f * CHUNK
        pltpu.sync_copy(ones_v, acc.at[pl.ds(r0, CHUNK)])

      def ofill(i, _):
        for j in range(128 // 16):
          ones_v[i, pl.ds(16 * j, 16)] = jnp.ones((16,), F32)
        return 0
      lax.fori_loop(0, CHUNK, ofill, 0)
      pltpu.sync_copy(idx_hbm.at[wid], didx)
      plsc.subcore_barrier()

      def body(k, _):
        pltpu.async_copy(ones_v, acc.at[didx.at[k]], sem, add=True)
        return 0
      lax.fori_loop(0, NCH, body, 0)

      def drain(k, _):
        pltpu.make_async_copy(ones_v, acc.at[didx.at[0]], sem).wait()
        return 0
      lax.fori_loop(0, NCH, drain, 0)
      plsc.subcore_barrier()

      for f in range(NFLUSH):
        r0 = s * ROWS_PER_SUB + f * CHUNK
        pltpu.sync_copy(acc.at[pl.ds(r0, CHUNK)], ones_v)
        pltpu.sync_copy(ones_v, out.at[d, c, pl.ds(r0, CHUNK)])

  return deg(src_idx, dst_idx)


def _agg_call(hw, src_idx, dst_idx, NB, FB):
  fo = NB * FB

  @functools.partial(
      pl.kernel,
      mesh=_mesh(),
      out_type=jax.ShapeDtypeStruct((2, NPAD, fo), F32),
      scratch_types=[
          pltpu.VMEM((NCH, CHUNK), jnp.int32),
          pltpu.VMEM((NCH, CHUNK), jnp.int32),
          pltpu.VMEM((CHUNK, FB), F32),
          pltpu.VMEM((CHUNK, FB), F32),
          pltpu.VMEM_SHARED((NPAD, FB), F32),
          pltpu.SemaphoreType.DMA,
          pltpu.SemaphoreType.DMA,
          pltpu.SemaphoreType.DMA,
          pltpu.SemaphoreType.DMA,
      ],
  )
  def agg(hw_hbm, src_hbm, dst_hbm, out, sidx, didx, buf0, buf1, acc,
          sem0, sem1, ssem0, ssem1):
    c = lax.axis_index("c")
    s = lax.axis_index("s")
    wid = c * NSUB + s
    bufs = (buf0, buf1)
    sems = (sem0, sem1)
    ssems = (ssem0, ssem1)

    def fill(i, _):
      for j in range(FB // 16):
        buf0[i, pl.ds(16 * j, 16)] = jnp.zeros((16,), F32)
      return 0

    pltpu.sync_copy(src_hbm.at[wid], sidx)
    pltpu.sync_copy(dst_hbm.at[wid], didx)

    for b in range(NB):
      lax.fori_loop(0, CHUNK, fill, 0)
      for f in range(NFLUSH):
        pltpu.sync_copy(buf0, acc.at[pl.ds(s * ROWS_PER_SUB + f * CHUNK, CHUNK)])
      plsc.subcore_barrier()

      tbl = hw_hbm.at[b]
      pltpu.make_async_copy(tbl.at[sidx.at[0]], buf0, sem0).start()

      def body(t, _):
        for par in range(2):
          k = 2 * t + par
          nk = k + 1
          other = (par + 1) % 2

          @pl.when(nk < NCH)
          def _():
            pltpu.make_async_copy(
                tbl.at[sidx.at[nk]], bufs[other], sems[other]
            ).start()

          pltpu.make_async_copy(tbl.at[sidx.at[k]], bufs[par], sems[par]).wait()
          pltpu.sync_copy(bufs[par], acc.at[didx.at[k]], add=True)
        return 0
      lax.fori_loop(0, NCH // 2, body, 0)
      plsc.subcore_barrier()

      for f in range(NFLUSH):
        r0 = s * ROWS_PER_SUB + f * CHUNK
        pltpu.sync_copy(acc.at[pl.ds(r0, CHUNK)], buf0)
        pltpu.sync_copy(buf0, out.at[c, pl.ds(r0, CHUNK), pl.ds(b * FB, FB)])
      if b + 1 < NB:
        plsc.subcore_barrier()

  return agg(hw, src_idx, dst_idx)


def _score_gather_call(ab, idx4):
  NCH_S = EPPAD // (NTILES * CHUNK)

  @functools.partial(
      pl.kernel,
      mesh=_mesh(),
      out_type=jax.ShapeDtypeStruct((4, EPPAD, 128), F32),
      scratch_types=[
          pltpu.VMEM((NCH_S, CHUNK), jnp.int32),
          pltpu.VMEM((CHUNK, 128), F32),
          pltpu.VMEM((CHUNK, 128), F32),
          pltpu.SemaphoreType.DMA,
          pltpu.SemaphoreType.DMA,
      ],
  )
  def sg(ab_hbm, idx_hbm, out, idxv, buf0, buf1, sem0, sem1):
    c = lax.axis_index("c")
    s = lax.axis_index("s")
    wid = c * NSUB + s
    bufs = (buf0, buf1)
    sems = (sem0, sem1)
    for t in range(4):
      tbl = ab_hbm.at[t % 2]
      pltpu.sync_copy(idx_hbm.at[t, wid], idxv)
      pltpu.make_async_copy(tbl.at[idxv.at[0]], buf0, sem0).start()
      for k in range(NCH_S):
        if k + 1 < NCH_S:
          pltpu.make_async_copy(
              tbl.at[idxv.at[k + 1]], bufs[(k + 1) % 2], sems[(k + 1) % 2]
          ).start()
        pltpu.make_async_copy(tbl.at[idxv.at[k]], bufs[k % 2], sems[k % 2]).wait()
        pltpu.sync_copy(
            bufs[k % 2], out.at[t, pl.ds(wid * 640 + k * CHUNK, CHUNK)]
        )

  return sg(ab, idx4)


def _isd_call(deg):
  def body(d_ref, o_ref):
    d = d_ref[...]
    o_ref[...] = lax.rsqrt(jnp.maximum(d[:, 0] + d[:, 1], 1.0))

  return pl.pallas_call(
      body,
      grid=(1,),
      in_specs=[pl.BlockSpec((2, 2, 80, 128), lambda i: (0, 0, 0, 0))],
      out_specs=pl.BlockSpec((2, 80, 128), lambda i: (0, 0, 0)),
      out_shape=jax.ShapeDtypeStruct((2, 80, 128), F32),
  )(deg)


def _mm_first_call(x, isd_out, W, NB, FB):
  fi = x.shape[1]

  def body(x_ref, so_ref, w_ref, o_ref):
    o_ref[0] = jnp.dot(x_ref[...], w_ref[...],
                       preferred_element_type=F32) * so_ref[...]

  return pl.pallas_call(
      body,
      grid=(N // BM, NB),
      in_specs=[
          pl.BlockSpec((BM, fi), lambda i, j: (i, 0)),
          pl.BlockSpec((BM, 1), lambda i, j: (i, 0)),
          pl.BlockSpec((fi, FB), lambda i, j: (0, j)),
      ],
      out_specs=pl.BlockSpec((1, BM, FB), lambda i, j: (j, i, 0)),
      out_shape=jax.ShapeDtypeStruct((NB, N, FB), F32),
  )(x, isd_out, W)


def _mm_mid_call(acc, isd_in, b, isd_out, W, NB, FB, relu):
  fi = acc.shape[2]

  def body(a_ref, si_ref, b_ref, so_ref, w_ref, o_ref, a_s):
    j = pl.program_id(1)

    @pl.when(j == 0)
    def _():
      h = (a_ref[0] + a_ref[1]) * si_ref[...] + b_ref[...]
      if relu:
        h = jnp.maximum(h, 0.0)
      a_s[...] = h

    o_ref[0] = jnp.dot(a_s[...], w_ref[...],
                       preferred_element_type=F32) * so_ref[...]

  return pl.pallas_call(
      body,
      grid=(N // BM, NB),
      in_specs=[
          pl.BlockSpec((2, BM, fi), lambda i, j: (0, i, 0)),
          pl.BlockSpec((BM, 1), lambda i, j: (i, 0)),
          pl.BlockSpec((1, fi), lambda i, j: (0, 0)),
          pl.BlockSpec((BM, 1), lambda i, j: (i, 0)),
          pl.BlockSpec((fi, FB), lambda i, j: (0, j)),
      ],
      out_specs=pl.BlockSpec((1, BM, FB), lambda i, j: (j, i, 0)),
      out_shape=jax.ShapeDtypeStruct((NB, N, FB), F32),
      scratch_shapes=[pltpu.VMEM((BM, fi), F32)],
  )(acc, isd_in, b, isd_out, W)


def _mm_ab_call(acc, isd_in, b4, Wp, bb):
  def body(a_ref, si_ref, b_ref, w_ref, bb_ref, o_ref, a_s):
    j = pl.program_id(1)

    @pl.when(j == 0)
    def _():
      a_s[...] = (a_ref[0, :, :64] + a_ref[1, :, :64]) * si_ref[...] + b_ref[...]

    o_ref[0] = jnp.dot(a_s[...], w_ref[0], preferred_element_type=F32) + bb_ref[0]

  return pl.pallas_call(
      body,
      grid=(N // BM, 2),
      in_specs=[
          pl.BlockSpec((2, BM, 128), lambda i, j: (0, i, 0)),
          pl.BlockSpec((BM, 1), lambda i, j: (i, 0)),
          pl.BlockSpec((1, 64), lambda i, j: (0, 0)),
          pl.BlockSpec((1, 64, 128), lambda i, j: (j, 0, 0)),
          pl.BlockSpec((1, 1, 128), lambda i, j: (j, 0, 0)),
      ],
      out_specs=pl.BlockSpec((1, BM, 128), lambda i, j: (j, i, 0)),
      out_shape=jax.ShapeDtypeStruct((2, N, 128), F32),
      scratch_shapes=[pltpu.VMEM((BM, 64), F32)],
  )(acc, isd_in, b4, Wp, bb)


def _score_mm_call(Z, Wp2, bp2):
  BM2 = 1024

  def body(z1_ref, z2_ref, w_ref, b_ref, o_ref):
    z = jnp.maximum(z1_ref[0, :, :64] + z2_ref[0, :, :64], 0.0)
    o_ref[0] = jnp.dot(z, w_ref[...], preferred_element_type=F32) + b_ref[...]

  return pl.pallas_call(
      body,
      grid=(2, EPPAD // BM2),
      in_specs=[
          pl.BlockSpec((1, BM2, 128), lambda k, i: (2 * k, i, 0)),
          pl.BlockSpec((1, BM2, 128), lambda k, i: (2 * k + 1, i, 0)),
          pl.BlockSpec((64, 1), lambda k, i: (0, 0)),
          pl.BlockSpec((1, 1), lambda k, i: (0, 0)),
      ],
      out_specs=pl.BlockSpec((1, BM2, 1), lambda k, i: (k, i, 0)),
      out_shape=jax.ShapeDtypeStruct((2, EPPAD, 1), F32),
  )(Z, Z, Wp2, bp2)


def _pad_gather(idx, total):
  pad = total - idx.shape[0]
  fill = (jnp.arange(pad, dtype=jnp.int32) * 97) % N
  return jnp.concatenate([idx.astype(jnp.int32), fill])


def _pad_trash(idx, total):
  pad = total - idx.shape[0]
  fill = N + (jnp.arange(pad, dtype=jnp.int32) % (NPAD - N))
  return jnp.concatenate([idx.astype(jnp.int32), fill])


def kernel(x, edge_index, pos_edge_index, neg_edge_index,
           W1, b1, W2, b2, W3, b3, W4, b4, Wp1, bp1, Wp2, bp2):
  src, dst = edge_index[0], edge_index[1]
  src_g = _pad_gather(src, EPAD).reshape(NTILES, NCH, CHUNK)
  dst_s = _pad_trash(dst, EPAD).reshape(NTILES, NCH, CHUNK)
  src_t = _pad_trash(src, EPAD).reshape(NTILES, NCH, CHUNK)

  deg = _deg_call(src_t, dst_s)
  isd = _isd_call(deg[..., 0].reshape(2, 2, 80, 128))
  isd_out = isd[0].reshape(NPAD, 1)[:N]
  isd_in = isd[1].reshape(NPAD, 1)[:N]

  hw1 = _mm_first_call(x, isd_out, W1, 4, 128)
  acc1 = _agg_call(hw1, src_g, dst_s, 4, 128)
  hw2 = _mm_mid_call(acc1[:, :N], isd_in, b1.reshape(1, -1), isd_out, W2,
                     2, 128, True)
  acc2 = _agg_call(hw2, src_g, dst_s, 2, 128)
  hw3 = _mm_mid_call(acc2[:, :N], isd_in, b2.reshape(1, -1), isd_out, W3,
                     1, 128, True)
  acc3 = _agg_call(hw3, src_g, dst_s, 1, 128)
  W4p = jnp.pad(W4, ((0, 0), (0, 64)))
  hw4 = _mm_mid_call(acc3[:, :N], isd_in, b3.reshape(1, -1), isd_out, W4p,
                     1, 128, True)
  acc4 = _agg_call(hw4, src_g, dst_s, 1, 128)

  Wp = jnp.pad(Wp1.reshape(2, 64, 64), ((0, 0), (0, 0), (0, 64)))
  bb = jnp.pad(jnp.stack([bp1, jnp.zeros_like(bp1)]).reshape(2, 1, 64),
               ((0, 0), (0, 0), (0, 64)))
  AB = _mm_ab_call(acc4[:, :N], isd_in, b4.reshape(1, -1), Wp, bb)

  idx4 = jnp.stack([
      _pad_gather(pos_edge_index[0], EPPAD),
      _pad_gather(pos_edge_index[1], EPPAD),
      _pad_gather(neg_edge_index[0], EPPAD),
      _pad_gather(neg_edge_index[1], EPPAD),
  ]).reshape(4, NTILES, EPPAD // (NTILES * CHUNK), CHUNK)

  Z = _score_gather_call(AB, idx4)
  S = _score_mm_call(Z, Wp2, bp2.reshape(1, 1))
  return S[0, :EP], S[1, :EP]
